# Initial kernel scaffold; baseline (speedup 1.0000x reference)
import jax
import jax.numpy as jnp
from jax.experimental import pallas as pl


def kernel(target, context, negative, in_embed, out_embed):
    raise NotImplementedError("write your pallas kernel here")


# speedup vs baseline: 4.4298x; 4.4298x over previous
import functools

import jax
import jax.numpy as jnp
from jax import lax
from jax.experimental import pallas as pl
from jax.experimental.pallas import tpu as pltpu
from jax.experimental.pallas import tpu_sc as plsc

_VOCAB = 1000000
_EMB = 64
_B = 16384
_NEG = 20
_NSCORE = _NEG + 1

_NW = 32
_IPW = _B // _NW
_C = 64
_NCH = _IPW // _C
_NROWS = _C * _NEG
_NIDX_R = _NROWS // 128
_SLEN = _C * _NSCORE


def _sc_scores(target, context, neg2d, in_embed, out_embed):
    mesh = plsc.VectorSubcoreMesh(core_axis_name="c", subcore_axis_name="s")

    @functools.partial(
        pl.kernel,
        mesh=mesh,
        compiler_params=pltpu.CompilerParams(
            needs_layout_passes=False, use_tc_tiling_on_sc=False),
        out_type=jax.ShapeDtypeStruct((_B * _NSCORE,), jnp.float32),
        scratch_types=[
            pltpu.VMEM((_C,), jnp.int32),
            pltpu.VMEM((_C,), jnp.int32),
            pltpu.VMEM((_NROWS,), jnp.int32),
            pltpu.VMEM((_C, _EMB), jnp.float32),
            pltpu.VMEM((_C, _EMB), jnp.float32),
            pltpu.VMEM((_NROWS, _EMB), jnp.float32),
            pltpu.VMEM((_SLEN,), jnp.float32),
            pltpu.SemaphoreType.DMA,
        ],
    )
    def scores_kernel(tgt_h, ctx_h, neg_h, ine_h, oute_h, out_h,
                      tidx, cidx, nidx, vbuf, ubuf, nbuf, sbuf, sem):
        wid = lax.axis_index("s") * 2 + lax.axis_index("c")

        def chunk_body(g, carry):
            base = wid * _IPW + g * _C
            pltpu.sync_copy(tgt_h.at[pl.ds(base, _C)], tidx)
            pltpu.sync_copy(ctx_h.at[pl.ds(base, _C)], cidx)
            pltpu.sync_copy(neg_h.at[pl.ds(base * _NEG, _NROWS)], nidx)
            copies = [
                pltpu.async_copy(ine_h.at[tidx], vbuf, sem),
                pltpu.async_copy(oute_h.at[cidx], ubuf, sem),
            ]
            for j in range(_NIDX_R):
                copies.append(pltpu.async_copy(
                    oute_h.at[nidx.at[pl.ds(j * 128, 128)]],
                    nbuf.at[pl.ds(j * 128, 128)], sem))
            for cp in copies:
                cp.wait()

            lane0 = lax.iota(jnp.int32, 16) == 0

            def put(pos, s):
                plsc.store_scatter(
                    sbuf, [jnp.full((16,), pos, jnp.int32)],
                    jnp.full((16,), s, jnp.float32), mask=lane0)

            def item_body(i, carry2):
                va = [vbuf[i, pl.ds(16 * t, 16)] for t in range(4)]
                p = va[0] * ubuf[i, pl.ds(0, 16)]
                for t in range(1, 4):
                    p = p + va[t] * ubuf[i, pl.ds(16 * t, 16)]
                put(i * _NSCORE, jnp.sum(p))
                for kk in range(_NEG):
                    r = i * _NEG + kk
                    q = va[0] * nbuf[r, pl.ds(0, 16)]
                    for t in range(1, 4):
                        q = q + va[t] * nbuf[r, pl.ds(16 * t, 16)]
                    put(i * _NSCORE + 1 + kk, -jnp.sum(q))
                return carry2

            lax.fori_loop(0, _C, item_body, 0)
            pltpu.sync_copy(sbuf, out_h.at[pl.ds(base * _NSCORE, _SLEN)])
            return carry

        lax.fori_loop(0, _NCH, chunk_body, 0)

    return scores_kernel(target, context, neg2d, in_embed, out_embed)


def _loss_body(x_ref, o_ref):
    x = x_ref[...]
    ls = jnp.minimum(x, 0.0) - jnp.log1p(jnp.exp(-jnp.abs(x)))
    o_ref[0, 0] = -jnp.sum(ls) / _B


def kernel(target, context, negative, in_embed, out_embed):
    negflat = negative.reshape(_B * _NEG)
    scores = _sc_scores(target, context, negflat, in_embed, out_embed)
    x2 = scores.reshape(_B * _NSCORE // 128, 128)
    out = pl.pallas_call(
        _loss_body,
        out_shape=jax.ShapeDtypeStruct((1, 1), jnp.float32),
        out_specs=pl.BlockSpec(memory_space=pltpu.SMEM),
    )(x2)
    return out[0, 0]


# speedup vs baseline: 4.8646x; 1.0982x over previous
import functools

import jax
import jax.numpy as jnp
from jax import lax
from jax.experimental import pallas as pl
from jax.experimental.pallas import tpu as pltpu
from jax.experimental.pallas import tpu_sc as plsc

_VOCAB = 1000000
_EMB = 64
_B = 16384
_NEG = 20
_NSCORE = _NEG + 1

_NW = 32
_IPW = _B // _NW
_C = 32
_NCH = _IPW // _C
_NROWS = _C * _NEG
_NIDX_R = _NROWS // 128
_SLEN = _C * _NSCORE


def _sc_scores(target, context, neg_flat, in_embed, out_embed):
    mesh = plsc.VectorSubcoreMesh(core_axis_name="c", subcore_axis_name="s")

    buf = lambda shape, dt: [pltpu.VMEM(shape, dt) for _ in range(2)]
    @functools.partial(
        pl.kernel,
        mesh=mesh,
        compiler_params=pltpu.CompilerParams(
            needs_layout_passes=False, use_tc_tiling_on_sc=False),
        out_type=jax.ShapeDtypeStruct((_B * _NSCORE,), jnp.float32),
        scratch_types=[
            buf((_C,), jnp.int32),
            buf((_C,), jnp.int32),
            buf((_NROWS,), jnp.int32),
            buf((_C, _EMB), jnp.float32),
            buf((_C, _EMB), jnp.float32),
            buf((_NROWS, _EMB), jnp.float32),
            pltpu.VMEM((_SLEN,), jnp.float32),
            [pltpu.SemaphoreType.DMA for _ in range(2)],
        ],
    )
    def scores_kernel(tgt_h, ctx_h, neg_h, ine_h, oute_h, out_h,
                      tidx, cidx, nidx, vbuf, ubuf, nbuf, sbuf, sems):
        wid = lax.axis_index("s") * 2 + lax.axis_index("c")
        lane15 = lax.iota(jnp.int32, 16) == 15

        def fire(g, p):
            base = wid * _IPW + g * _C
            pltpu.sync_copy(tgt_h.at[pl.ds(base, _C)], tidx[p])
            pltpu.sync_copy(ctx_h.at[pl.ds(base, _C)], cidx[p])
            pltpu.sync_copy(neg_h.at[pl.ds(base * _NEG, _NROWS)], nidx[p])
            cps = [
                pltpu.async_copy(ine_h.at[tidx[p]], vbuf[p], sems[p]),
                pltpu.async_copy(oute_h.at[cidx[p]], ubuf[p], sems[p]),
            ]
            for j in range(_NIDX_R):
                cps.append(pltpu.async_copy(
                    oute_h.at[nidx[p].at[pl.ds(j * 128, 128)]],
                    nbuf[p].at[pl.ds(j * 128, 128)], sems[p]))
            return cps

        def compute(g, p):
            base = wid * _IPW + g * _C

            def put(pos, vec):
                plsc.store_scatter(
                    sbuf, [jnp.full((16,), pos, jnp.int32)], vec, mask=lane15)

            def item_body(i, carry):
                va = [vbuf[p][i, pl.ds(16 * t, 16)] for t in range(4)]
                nva = [0.0 - va[t] for t in range(4)]
                q = va[0] * ubuf[p][i, pl.ds(0, 16)]
                for t in range(1, 4):
                    q = q + va[t] * ubuf[p][i, pl.ds(16 * t, 16)]
                put(i * _NSCORE, plsc.cumsum(q))
                for kk in range(_NEG):
                    r = i * _NEG + kk
                    q = nva[0] * nbuf[p][r, pl.ds(0, 16)]
                    for t in range(1, 4):
                        q = q + nva[t] * nbuf[p][r, pl.ds(16 * t, 16)]
                    put(i * _NSCORE + 1 + kk, plsc.cumsum(q))
                return carry

            lax.fori_loop(0, _C, item_body, 0)
            pltpu.sync_copy(sbuf, out_h.at[pl.ds(base * _NSCORE, _SLEN)])

        pending = fire(0, 0)
        for g in range(_NCH):
            p = g % 2
            if g + 1 < _NCH:
                nxt = fire(g + 1, 1 - p)
            else:
                nxt = []
            for cp in pending:
                cp.wait()
            compute(g, p)
            pending = nxt

    return scores_kernel(target, context, neg_flat, in_embed, out_embed)


def _loss_body(x_ref, o_ref):
    x = x_ref[...]
    ls = jnp.minimum(x, 0.0) - jnp.log1p(jnp.exp(-jnp.abs(x)))
    o_ref[0, 0] = -jnp.sum(ls) / _B


def kernel(target, context, negative, in_embed, out_embed):
    negflat = negative.reshape(_B * _NEG)
    scores = _sc_scores(target, context, negflat, in_embed, out_embed)
    x2 = scores.reshape(_B * _NSCORE // 128, 128)
    out = pl.pallas_call(
        _loss_body,
        out_shape=jax.ShapeDtypeStruct((1, 1), jnp.float32),
        out_specs=pl.BlockSpec(memory_space=pltpu.SMEM),
    )(x2)
    return out[0, 0]
